# Initial kernel scaffold; baseline (speedup 1.0000x reference)
import jax
import jax.numpy as jnp
from jax.experimental import pallas as pl


def kernel(z, y, protos, init_mask):
    raise NotImplementedError("write your pallas kernel here")


# speedup vs baseline: 2.3396x; 2.3396x over previous
import functools

import jax
import jax.numpy as jnp
from jax import lax
from jax.experimental import pallas as pl
from jax.experimental.pallas import tpu as pltpu
from jax.experimental.pallas import tpu_sc as plsc

_C = 8192
_D = 256
_N = 65536
_M = 0.99

_NS = 16
_DH = _D // 2
_B = 128
_NCHUNK = _N // (_NS * _B)
_RPT = _C // _NS


def _make_sums(cofs: int, with_ind: bool):
    mesh = plsc.VectorSubcoreMesh(core_axis_name="c", subcore_axis_name="s",
                                  num_cores=1)
    out_type = [jax.ShapeDtypeStruct((_C, _DH), jnp.float32)]
    scratch = [
        pltpu.VMEM_SHARED((_C, _DH), jnp.float32),
        pltpu.VMEM((2, _B, _DH), jnp.float32),
        pltpu.VMEM((8, 128), jnp.int32),
    ]
    if with_ind:
        out_type.append(
            jax.ShapeDtypeStruct((_NS, _C // 128, 128), jnp.float32))
        scratch.append(pltpu.VMEM((_C // 128, 128), jnp.float32))

    def body(*refs):
        if with_ind:
            (z_hbm, y_hbm, sums_hbm, ind_hbm,
             acc_sh, zbuf, ybuf, hist) = refs
        else:
            z_hbm, y_hbm, sums_hbm, acc_sh, zbuf, ybuf = refs
        tid = lax.axis_index("s")
        zeros16 = jnp.zeros((16,), jnp.float32)
        ones16 = jnp.ones((16,), jnp.float32)

        def _zrow_body(i, carry):
            for j in range(_DH // 16):
                zbuf[0, i, pl.ds(j * 16, 16)] = zeros16
            return carry

        lax.fori_loop(0, _B, _zrow_body, 0)
        if with_ind:
            def _zhist_body(i, carry):
                for j in range(128 // 16):
                    hist[i, pl.ds(j * 16, 16)] = zeros16
                return carry

            lax.fori_loop(0, _C // 128, _zhist_body, 0)

        r0 = pl.multiple_of(tid * _RPT, _RPT)
        for q in range(_RPT // _B):
            pltpu.sync_copy(zbuf.at[0], acc_sh.at[pl.ds(r0 + q * _B, _B)])
        plsc.subcore_barrier()

        for k in range(_NCHUNK):
            base = pl.multiple_of((tid * _NCHUNK + k) * _B, _B)
            if k % 8 == 0:
                pltpu.sync_copy(
                    y_hbm.at[pl.ds(pl.multiple_of(base // 128, 8), 8)], ybuf)
            pltpu.sync_copy(z_hbm.at[pl.ds(base, _B), pl.ds(cofs, _DH)],
                            zbuf.at[0])
            row = k % 8
            pltpu.sync_copy(zbuf.at[0], acc_sh.at[ybuf.at[row]], add=True)
            if with_ind:
                for l in range(128 // 16):
                    v = ybuf[row, pl.ds(l * 16, 16)]
                    plsc.store_scatter(
                        hist,
                        [lax.shift_right_logical(v, 7),
                         lax.bitwise_and(v, 127)],
                        ones16)

        plsc.subcore_barrier()
        for q in range(_RPT // _B):
            rq = r0 + q * _B
            pltpu.sync_copy(acc_sh.at[pl.ds(rq, _B)], zbuf.at[1])
            pltpu.sync_copy(zbuf.at[1], sums_hbm.at[pl.ds(rq, _B)])
        if with_ind:
            pltpu.sync_copy(hist, ind_hbm.at[tid])

    return pl.kernel(body, out_type=tuple(out_type), mesh=mesh,
                     scratch_types=scratch,
                     compiler_params=pltpu.CompilerParams(
                         needs_layout_passes=False))


_sums_lo = _make_sums(0, True)
_sums_hi = _make_sums(_DH, False)


def _fin_body(s0_ref, s1_ref, ind_ref, protos_ref, mask_ref, out_ref):
    present = ind_ref[...] > 0
    sums = jnp.concatenate([s0_ref[...], s1_ref[...]], axis=1)
    n1 = jnp.sqrt(jnp.sum(sums * sums, axis=1, keepdims=True))
    zc = sums / jnp.maximum(n1, 1e-12)
    p = protos_ref[...]
    ema = _M * p + (1.0 - _M) * zc
    n2 = jnp.sqrt(jnp.sum(ema * ema, axis=1, keepdims=True))
    ema = ema / jnp.maximum(n2, 1e-12)
    new = jnp.where(mask_ref[...] > 0, ema, zc)
    out_ref[...] = jnp.where(present, new, p)


_FIN_ROWS = 512


def _finalize(s0, s1, ind, protos, mask2):
    return pl.pallas_call(
        _fin_body,
        out_shape=jax.ShapeDtypeStruct((_C, _D), jnp.float32),
        grid=(_C // _FIN_ROWS,),
        in_specs=[
            pl.BlockSpec((_FIN_ROWS, _DH), lambda i: (i, 0)),
            pl.BlockSpec((_FIN_ROWS, _DH), lambda i: (i, 0)),
            pl.BlockSpec((_FIN_ROWS, 1), lambda i: (i, 0)),
            pl.BlockSpec((_FIN_ROWS, _D), lambda i: (i, 0)),
            pl.BlockSpec((_FIN_ROWS, 1), lambda i: (i, 0)),
        ],
        out_specs=pl.BlockSpec((_FIN_ROWS, _D), lambda i: (i, 0)),
    )(s0, s1, ind, protos, mask2)


def kernel(z, y, protos, init_mask):
    assert z.shape == (_N, _D) and protos.shape == (_C, _D)
    zf = z.astype(jnp.float32)
    y2 = y.astype(jnp.int32).reshape(_N // 128, 128)
    s0, ind16 = _sums_lo(zf, y2)
    (s1,) = _sums_hi(zf, y2)
    ind = ind16.sum(axis=0).reshape(_C, 1)
    mask2 = init_mask.reshape(_C, 1).astype(jnp.float32)
    return _finalize(s0, s1, ind, protos.astype(jnp.float32), mask2)


# speedup vs baseline: 3.3306x; 1.4236x over previous
import functools

import jax
import jax.numpy as jnp
from jax import lax
from jax.experimental import pallas as pl
from jax.experimental.pallas import tpu as pltpu
from jax.experimental.pallas import tpu_sc as plsc

_C = 8192
_D = 256
_N = 65536
_M = 0.99

_NS = 16
_DH = _D // 2
_B = 128
_NCHUNK = _N // (_NS * _B)
_RPT = _C // _NS


def _make_sums(cofs: int, with_ind: bool):
    mesh = plsc.VectorSubcoreMesh(core_axis_name="c", subcore_axis_name="s",
                                  num_cores=1)
    out_type = [jax.ShapeDtypeStruct((_C, _DH), jnp.float32)]
    scratch = [
        pltpu.VMEM_SHARED((_C, _DH), jnp.float32),
        pltpu.VMEM((2, _B, _DH), jnp.float32),
        pltpu.VMEM((2, 8, 128), jnp.int32),
        pltpu.SemaphoreType.DMA,
        pltpu.SemaphoreType.DMA,
        pltpu.SemaphoreType.DMA,
        pltpu.SemaphoreType.DMA,
    ]
    if with_ind:
        out_type.append(
            jax.ShapeDtypeStruct((_NS, _C // 128, 128), jnp.float32))
        scratch.append(pltpu.VMEM((_C // 128, 128), jnp.float32))

    def body(*refs):
        if with_ind:
            (z_hbm, y_hbm, sums_hbm, ind_hbm,
             acc_sh, zbuf, ybuf, zs0, zs1, ss0, ss1, hist) = refs
        else:
            (z_hbm, y_hbm, sums_hbm,
             acc_sh, zbuf, ybuf, zs0, zs1, ss0, ss1) = refs
        zsem = (zs0, zs1)
        ssem = (ss0, ss1)
        tid = lax.axis_index("s")
        zeros16 = jnp.zeros((16,), jnp.float32)
        ones16 = jnp.ones((16,), jnp.float32)

        def _zrow_body(i, carry):
            for j in range(_DH // 16):
                zbuf[0, i, pl.ds(j * 16, 16)] = zeros16
            return carry

        lax.fori_loop(0, _B, _zrow_body, 0)
        if with_ind:
            def _zhist_body(i, carry):
                for j in range(128 // 16):
                    hist[i, pl.ds(j * 16, 16)] = zeros16
                return carry

            lax.fori_loop(0, _C // 128, _zhist_body, 0)

        r0 = pl.multiple_of(tid * _RPT, _RPT)
        for q in range(_RPT // _B):
            pltpu.sync_copy(zbuf.at[0], acc_sh.at[pl.ds(r0 + q * _B, _B)])
        plsc.subcore_barrier()

        def _yload(grp):
            base = pl.multiple_of((tid * _NCHUNK + grp * 8) * _B, _B)
            pltpu.sync_copy(
                y_hbm.at[pl.ds(pl.multiple_of(base // 128, 8), 8)],
                ybuf.at[grp % 2])

        def _zstart(k):
            b = k % 2
            base = pl.multiple_of((tid * _NCHUNK + k) * _B, _B)
            return pltpu.async_copy(
                z_hbm.at[pl.ds(base, _B), pl.ds(cofs, _DH)],
                zbuf.at[b], zsem[b])

        desc_s = [None, None]
        _yload(0)
        desc_z = [_zstart(0), None]
        for k in range(_NCHUNK):
            b = k % 2
            nb = 1 - b
            if k + 1 < _NCHUNK:
                if (k + 1) % 8 == 0:
                    _yload((k + 1) // 8)
                if desc_s[nb] is not None:
                    desc_s[nb].wait()
                desc_z[nb] = _zstart(k + 1)
            desc_z[b].wait()
            yb = (k // 8) % 2
            row = k % 8
            desc_s[b] = pltpu.async_copy(
                zbuf.at[b], acc_sh.at[ybuf.at[yb, row]], ssem[b], add=True)
            if with_ind:
                for l in range(128 // 16):
                    v = ybuf[yb, row, pl.ds(l * 16, 16)]
                    plsc.store_scatter(
                        hist,
                        [lax.shift_right_logical(v, 7),
                         lax.bitwise_and(v, 127)],
                        ones16)
        desc_s[0].wait()
        desc_s[1].wait()

        plsc.subcore_barrier()
        for q in range(_RPT // _B):
            rq = r0 + q * _B
            pltpu.sync_copy(acc_sh.at[pl.ds(rq, _B)], zbuf.at[1])
            pltpu.sync_copy(zbuf.at[1], sums_hbm.at[pl.ds(rq, _B)])
        if with_ind:
            pltpu.sync_copy(hist, ind_hbm.at[tid])

    return pl.kernel(body, out_type=tuple(out_type), mesh=mesh,
                     scratch_types=scratch,
                     compiler_params=pltpu.CompilerParams(
                         needs_layout_passes=False))


_sums_lo = _make_sums(0, True)
_sums_hi = _make_sums(_DH, False)


def _fin_body(s0_ref, s1_ref, ind_ref, protos_ref, mask_ref, out_ref):
    present = ind_ref[...] > 0
    sums = jnp.concatenate([s0_ref[...], s1_ref[...]], axis=1)
    n1 = jnp.sqrt(jnp.sum(sums * sums, axis=1, keepdims=True))
    zc = sums / jnp.maximum(n1, 1e-12)
    p = protos_ref[...]
    ema = _M * p + (1.0 - _M) * zc
    n2 = jnp.sqrt(jnp.sum(ema * ema, axis=1, keepdims=True))
    ema = ema / jnp.maximum(n2, 1e-12)
    new = jnp.where(mask_ref[...] > 0, ema, zc)
    out_ref[...] = jnp.where(present, new, p)


_FIN_ROWS = 512


def _finalize(s0, s1, ind, protos, mask2):
    return pl.pallas_call(
        _fin_body,
        out_shape=jax.ShapeDtypeStruct((_C, _D), jnp.float32),
        grid=(_C // _FIN_ROWS,),
        in_specs=[
            pl.BlockSpec((_FIN_ROWS, _DH), lambda i: (i, 0)),
            pl.BlockSpec((_FIN_ROWS, _DH), lambda i: (i, 0)),
            pl.BlockSpec((_FIN_ROWS, 1), lambda i: (i, 0)),
            pl.BlockSpec((_FIN_ROWS, _D), lambda i: (i, 0)),
            pl.BlockSpec((_FIN_ROWS, 1), lambda i: (i, 0)),
        ],
        out_specs=pl.BlockSpec((_FIN_ROWS, _D), lambda i: (i, 0)),
    )(s0, s1, ind, protos, mask2)


def kernel(z, y, protos, init_mask):
    assert z.shape == (_N, _D) and protos.shape == (_C, _D)
    zf = z.astype(jnp.float32)
    y2 = y.astype(jnp.int32).reshape(_N // 128, 128)
    s0, ind16 = _sums_lo(zf, y2)
    (s1,) = _sums_hi(zf, y2)
    ind = ind16.sum(axis=0).reshape(_C, 1)
    mask2 = init_mask.reshape(_C, 1).astype(jnp.float32)
    return _finalize(s0, s1, ind, protos.astype(jnp.float32), mask2)


# speedup vs baseline: 3.5064x; 1.0528x over previous
import functools

import jax
import jax.numpy as jnp
from jax import lax
from jax.experimental import pallas as pl
from jax.experimental.pallas import tpu as pltpu
from jax.experimental.pallas import tpu_sc as plsc

_C = 8192
_D = 256
_N = 65536
_M = 0.99

_NS = 16
_DH = _D // 2
_B = 128
_NCHUNK = _N // (_NS * _B)
_RPT = _C // _NS


def _make_sums(cofs: int, with_ind: bool):
    mesh = plsc.VectorSubcoreMesh(core_axis_name="c", subcore_axis_name="s",
                                  num_cores=1)
    out_type = [jax.ShapeDtypeStruct((_C, _DH), jnp.float32)]
    scratch = [
        pltpu.VMEM_SHARED((_C, _DH), jnp.float32),
        pltpu.VMEM((3, _B, _DH), jnp.float32),
        pltpu.VMEM((2, 8, 128), jnp.int32),
        pltpu.SemaphoreType.DMA,
        pltpu.SemaphoreType.DMA,
        pltpu.SemaphoreType.DMA,
        pltpu.SemaphoreType.DMA,
        pltpu.SemaphoreType.DMA,
        pltpu.SemaphoreType.DMA,
    ]
    if with_ind:
        out_type.append(
            jax.ShapeDtypeStruct((_NS, _C // 128, 128), jnp.float32))
        scratch.append(pltpu.VMEM((_C // 128, 128), jnp.float32))

    def body(*refs):
        if with_ind:
            (z_hbm, y_hbm, sums_hbm, ind_hbm,
             acc_sh, zbuf, ybuf, zs0, zs1, zs2, ss0, ss1, ss2, hist) = refs
        else:
            (z_hbm, y_hbm, sums_hbm,
             acc_sh, zbuf, ybuf, zs0, zs1, zs2, ss0, ss1, ss2) = refs
        zsem = (zs0, zs1, zs2)
        ssem = (ss0, ss1, ss2)
        tid = lax.axis_index("s")
        zeros16 = jnp.zeros((16,), jnp.float32)
        ones16 = jnp.ones((16,), jnp.float32)

        def _zrow_body(i, carry):
            for j in range(_DH // 16):
                zbuf[0, i, pl.ds(j * 16, 16)] = zeros16
            return carry

        lax.fori_loop(0, _B, _zrow_body, 0)
        if with_ind:
            def _zhist_body(i, carry):
                for j in range(128 // 16):
                    hist[i, pl.ds(j * 16, 16)] = zeros16
                return carry

            lax.fori_loop(0, _C // 128, _zhist_body, 0)

        r0 = pl.multiple_of(tid * _RPT, _RPT)
        for q in range(_RPT // _B):
            pltpu.sync_copy(zbuf.at[0], acc_sh.at[pl.ds(r0 + q * _B, _B)])
        plsc.subcore_barrier()

        def _yload(grp):
            base = pl.multiple_of((tid * _NCHUNK + grp * 8) * _B, _B)
            pltpu.sync_copy(
                y_hbm.at[pl.ds(pl.multiple_of(base // 128, 8), 8)],
                ybuf.at[grp % 2])

        def _zstart(k):
            b = k % 3
            base = pl.multiple_of((tid * _NCHUNK + k) * _B, _B)
            return pltpu.async_copy(
                z_hbm.at[pl.ds(base, _B), pl.ds(cofs, _DH)],
                zbuf.at[b], zsem[b])

        desc_s = [None, None, None]
        _yload(0)
        desc_z = [_zstart(0), _zstart(1), None]
        for k in range(_NCHUNK):
            b = k % 3
            kk = k + 2
            if kk < _NCHUNK:
                if kk % 8 == 0:
                    _yload(kk // 8)
                bb = kk % 3
                if desc_s[bb] is not None:
                    desc_s[bb].wait()
                desc_z[bb] = _zstart(kk)
            desc_z[b].wait()
            yb = (k // 8) % 2
            row = k % 8
            desc_s[b] = pltpu.async_copy(
                zbuf.at[b], acc_sh.at[ybuf.at[yb, row]], ssem[b], add=True)
            if with_ind:
                for l in range(128 // 16):
                    v = ybuf[yb, row, pl.ds(l * 16, 16)]
                    plsc.store_scatter(
                        hist,
                        [lax.shift_right_logical(v, 7),
                         lax.bitwise_and(v, 127)],
                        ones16)
        for b in range(3):
            if desc_s[b] is not None:
                desc_s[b].wait()

        plsc.subcore_barrier()
        for q in range(_RPT // _B):
            rq = r0 + q * _B
            pltpu.sync_copy(acc_sh.at[pl.ds(rq, _B)], zbuf.at[1])
            pltpu.sync_copy(zbuf.at[1], sums_hbm.at[pl.ds(rq, _B)])
        if with_ind:
            pltpu.sync_copy(hist, ind_hbm.at[tid])

    return pl.kernel(body, out_type=tuple(out_type), mesh=mesh,
                     scratch_types=scratch,
                     compiler_params=pltpu.CompilerParams(
                         needs_layout_passes=False))


_sums_lo = _make_sums(0, True)
_sums_hi = _make_sums(_DH, False)


def _fin_body(s0_ref, s1_ref, ind_ref, protos_ref, mask_ref, out_ref):
    present = ind_ref[...] > 0
    sums = jnp.concatenate([s0_ref[...], s1_ref[...]], axis=1)
    n1 = jnp.sqrt(jnp.sum(sums * sums, axis=1, keepdims=True))
    zc = sums / jnp.maximum(n1, 1e-12)
    p = protos_ref[...]
    ema = _M * p + (1.0 - _M) * zc
    n2 = jnp.sqrt(jnp.sum(ema * ema, axis=1, keepdims=True))
    ema = ema / jnp.maximum(n2, 1e-12)
    new = jnp.where(mask_ref[...] > 0, ema, zc)
    out_ref[...] = jnp.where(present, new, p)


_FIN_ROWS = 512


def _finalize(s0, s1, ind, protos, mask2):
    return pl.pallas_call(
        _fin_body,
        out_shape=jax.ShapeDtypeStruct((_C, _D), jnp.float32),
        grid=(_C // _FIN_ROWS,),
        in_specs=[
            pl.BlockSpec((_FIN_ROWS, _DH), lambda i: (i, 0)),
            pl.BlockSpec((_FIN_ROWS, _DH), lambda i: (i, 0)),
            pl.BlockSpec((_FIN_ROWS, 1), lambda i: (i, 0)),
            pl.BlockSpec((_FIN_ROWS, _D), lambda i: (i, 0)),
            pl.BlockSpec((_FIN_ROWS, 1), lambda i: (i, 0)),
        ],
        out_specs=pl.BlockSpec((_FIN_ROWS, _D), lambda i: (i, 0)),
    )(s0, s1, ind, protos, mask2)


def kernel(z, y, protos, init_mask):
    assert z.shape == (_N, _D) and protos.shape == (_C, _D)
    zf = z.astype(jnp.float32)
    y2 = y.astype(jnp.int32).reshape(_N // 128, 128)
    s0, ind16 = _sums_lo(zf, y2)
    (s1,) = _sums_hi(zf, y2)
    ind = ind16.sum(axis=0).reshape(_C, 1)
    mask2 = init_mask.reshape(_C, 1).astype(jnp.float32)
    return _finalize(s0, s1, ind, protos.astype(jnp.float32), mask2)
